# baseline (speedup 1.0000x reference)
import jax
import jax.numpy as jnp
from jax.experimental import pallas as pl
from jax.experimental.pallas import tpu as pltpu

_P = 98
_HALF = _P // 2
_KO = 2
_CH = _P // _KO


def _mix_kernel(lam_ref, idx_ref, xa_ref, xb_ref, o_hbm, oa, sems):
    i = pl.program_id(0)
    n = pl.num_programs(0)
    slot = i % 2

    def wait_out(s):
        for k in range(_KO):
            pltpu.make_async_copy(
                oa.at[s, pl.ds(k * _CH, _CH)],
                o_hbm.at[pl.ds(k * _CH, _CH)],
                sems.at[s, k],
            ).wait()

    @pl.when(i >= 2)
    def _():
        wait_out(slot)

    l = lam_ref[0]
    row = jax.lax.broadcasted_iota(jnp.int32, (128, 128), 0)
    col = jax.lax.broadcasted_iota(jnp.int32, (128, 128), 1)
    idx = idx_ref[...]
    a = (l * (row == col).astype(jnp.float32)
         + (1.0 - l) * (col == idx).astype(jnp.float32))
    for q in range(_HALF):
        oa[slot, q] = jnp.dot(a, xa_ref[q], preferred_element_type=jnp.float32)
    for q in range(_HALF):
        oa[slot, _HALF + q] = jnp.dot(a, xb_ref[q],
                                      preferred_element_type=jnp.float32)

    for k in range(_KO):
        pltpu.make_async_copy(
            oa.at[slot, pl.ds(k * _CH, _CH)],
            o_hbm.at[pl.ds(i * _P + k * _CH, _CH)],
            sems.at[slot, k],
        ).start()

    @pl.when(i == n - 1)
    def _():
        wait_out(1 - slot)
        wait_out(slot)


def kernel(x, lam, index):
    B, C, H, W = x.shape
    xt = jnp.transpose(x, (2, 3, 0, 1)).reshape(H * W, B, C)
    idx2d = index.astype(jnp.int32).reshape(B, 1)
    out = pl.pallas_call(
        _mix_kernel,
        grid_spec=pltpu.PrefetchScalarGridSpec(
            num_scalar_prefetch=1,
            grid=(H * W // _P,),
            in_specs=[
                pl.BlockSpec((B, 1), lambda i, lam_ref: (0, 0)),
                pl.BlockSpec((_HALF, B, C), lambda i, lam_ref: (2 * i, 0, 0)),
                pl.BlockSpec((_HALF, B, C), lambda i, lam_ref: (2 * i + 1, 0, 0)),
            ],
            out_specs=pl.BlockSpec(memory_space=pl.ANY),
            scratch_shapes=[
                pltpu.VMEM((2, _P, B, C), jnp.float32),
                pltpu.SemaphoreType.DMA((2, _KO)),
            ],
        ),
        out_shape=jax.ShapeDtypeStruct((H * W, B, C), x.dtype),
    )(lam, idx2d, xt, xt)
    return jnp.transpose(out.reshape(H, W, B, C), (2, 3, 0, 1))
